# baseline (speedup 1.0000x reference)
import functools

import jax
import jax.numpy as jnp
from jax import lax
from jax.experimental import pallas as pl
from jax.experimental.pallas import tpu as pltpu
from jax.experimental.pallas import tpu_sc as plsc

NUM_NODES = 1000000
NP = NUM_NODES // 2
B = 16384
D = 64
NT = 16
G = 128
CH = B // NT
NG = CH // G
BI = 1024
BJ = 1024
NBI = B // BI
NBJ = B // BJ


def _sc_body(ns2, pidx1, val128, out, *rest):
    pidx_g = rest[0:NG]
    big1, big2, sem_g = rest[NG:]

    c = lax.axis_index("c")
    s = lax.axis_index("s")

    for j in range(NG):
        pltpu.sync_copy(pidx1.at[pl.ds(s * CH + j * G, G)], pidx_g[j])

    for j in range(NG):
        base = s * CH + j * G
        pltpu.async_copy(ns2.at[pidx_g[j]], big1, sem_g).wait()
        pltpu.sync_copy(val128.at[pl.ds(base, G)], big2)

        def addrow(row, carry):
            for k in range(G // 16):
                d = pl.ds(k * 16, 16)
                big1[row, d] = big1[row, d] + big2[row, d]
            return carry
        lax.fori_loop(0, G, addrow, 0)

        @pl.when((base // (B // 2)) == c)
        def _(base=base):
            pltpu.sync_copy(big1, out.at[pl.ds(base, G)])


def _tc_body(idxi, idxjt, valj, vali, sc, o, acc):
    j = pl.program_id(1)

    @pl.when(j == 0)
    def _():
        acc[...] = jnp.zeros_like(acc)

    m = (idxi[:, :1] == idxjt[:1, :]).astype(jnp.bfloat16)
    acc[...] += lax.dot_general(
        m, valj[...].astype(jnp.bfloat16),
        (((1,), (0,)), ((), ())),
        preferred_element_type=jnp.float32)

    @pl.when(j == NBJ - 1)
    def _():
        vb = vali[...].astype(jnp.bfloat16).astype(jnp.float32)
        o[...] = sc[...] + acc[...] - vb


def kernel(node_states, idx, val):
    idx32 = idx.astype(jnp.int32)
    pidx1 = idx32 >> 1
    ns2 = node_states.reshape(NP, G)
    odd = (idx32 & 1).astype(jnp.bool_)
    zero = jnp.zeros_like(val)
    val128 = jnp.where(odd[:, None],
                       jnp.concatenate([zero, val], axis=1),
                       jnp.concatenate([val, zero], axis=1))

    sc_run = pl.kernel(
        _sc_body,
        out_type=jax.ShapeDtypeStruct((B, G), jnp.float32),
        mesh=plsc.VectorSubcoreMesh(core_axis_name="c", subcore_axis_name="s"),
        scratch_types=(
            [pltpu.VMEM((G,), jnp.int32)] * NG
            + [pltpu.VMEM((G, G), jnp.float32),
               pltpu.VMEM((G, G), jnp.float32),
               pltpu.SemaphoreType.DMA]
        ),
    )
    sc128 = sc_run(ns2, pidx1, val128)
    sc_sel = jnp.where(odd[:, None], sc128[:, D:], sc128[:, :D])

    idxf = idx32.astype(jnp.float32)
    idxi = jnp.broadcast_to(idxf[:, None], (B, 128))
    idxjt = jnp.broadcast_to(idxf[None, :], (128, B))

    tc_run = pl.pallas_call(
        _tc_body,
        grid=(NBI, NBJ),
        in_specs=[
            pl.BlockSpec((BI, 128), lambda i, j: (i, 0)),
            pl.BlockSpec((128, BJ), lambda i, j: (0, j)),
            pl.BlockSpec((BJ, D), lambda i, j: (j, 0)),
            pl.BlockSpec((BI, D), lambda i, j: (i, 0)),
            pl.BlockSpec((BI, D), lambda i, j: (i, 0)),
        ],
        out_specs=pl.BlockSpec((BI, D), lambda i, j: (i, 0)),
        out_shape=jax.ShapeDtypeStruct((B, D), jnp.float32),
        scratch_shapes=[pltpu.VMEM((BI, D), jnp.float32)],
        compiler_params=pltpu.CompilerParams(
            dimension_semantics=("parallel", "arbitrary")),
    )
    return tc_run(idxi, idxjt, val, val, sc_sel)


# speedup vs baseline: 1.0342x; 1.0342x over previous
import functools

import jax
import jax.numpy as jnp
from jax import lax
from jax.experimental import pallas as pl
from jax.experimental.pallas import tpu as pltpu
from jax.experimental.pallas import tpu_sc as plsc

NUM_NODES = 1000000
NP = NUM_NODES // 2
B = 16384
D = 64
NT = 16
G = 128
CH = B // NT
NG = CH // G
BI = 1024
BJ = 1024
NBI = B // BI
NBJ = B // BJ


def _sc_body(ns2, pidx1, val128, out, *rest):
    pidx_g = rest[0:NG]
    big1, big2, sem_g = rest[NG:]

    c = lax.axis_index("c")
    s = lax.axis_index("s")

    for j in range(NG):
        pltpu.sync_copy(pidx1.at[pl.ds(s * CH + j * G, G)], pidx_g[j])

    for j in range(NG):
        base = s * CH + j * G
        pltpu.async_copy(ns2.at[pidx_g[j]], big1, sem_g).wait()
        pltpu.sync_copy(val128.at[pl.ds(base, G)], big2)

        def addrow(row, carry):
            for k in range(G // 16):
                d = pl.ds(k * 16, 16)
                big1[row, d] = big1[row, d] + big2[row, d]
            return carry
        lax.fori_loop(0, G, addrow, 0)

        @pl.when((base // (B // 2)) == c)
        def _(base=base):
            pltpu.sync_copy(big1, out.at[pl.ds(base, G)])


def _tc_body(idxi, idxjc, valj, vali, sc, o, acc):
    j = pl.program_id(1)

    @pl.when(j == 0)
    def _():
        acc[...] = jnp.zeros_like(acc)

    ii = idxi[...]
    vj = valj[...].astype(jnp.bfloat16)
    upd = jnp.zeros_like(acc)
    for t in range(BJ // 128):
        m = (ii == idxjc[t:t + 1, :]).astype(jnp.bfloat16)
        upd += lax.dot_general(
            m, vj[t * 128:(t + 1) * 128, :],
            (((1,), (0,)), ((), ())),
            preferred_element_type=jnp.float32)
    acc[...] += upd

    @pl.when(j == NBJ - 1)
    def _():
        vb = vali[...].astype(jnp.bfloat16).astype(jnp.float32)
        o[...] = sc[...] + acc[...] - vb


def kernel(node_states, idx, val):
    idx32 = idx.astype(jnp.int32)
    pidx1 = idx32 >> 1
    ns2 = node_states.reshape(NP, G)
    odd = (idx32 & 1).astype(jnp.bool_)
    zero = jnp.zeros_like(val)
    val128 = jnp.where(odd[:, None],
                       jnp.concatenate([zero, val], axis=1),
                       jnp.concatenate([val, zero], axis=1))

    sc_run = pl.kernel(
        _sc_body,
        out_type=jax.ShapeDtypeStruct((B, G), jnp.float32),
        mesh=plsc.VectorSubcoreMesh(core_axis_name="c", subcore_axis_name="s"),
        scratch_types=(
            [pltpu.VMEM((G,), jnp.int32)] * NG
            + [pltpu.VMEM((G, G), jnp.float32),
               pltpu.VMEM((G, G), jnp.float32),
               pltpu.SemaphoreType.DMA]
        ),
    )
    sc128 = sc_run(ns2, pidx1, val128)
    sc_sel = jnp.where(odd[:, None], sc128[:, D:], sc128[:, :D])

    idxf = idx32.astype(jnp.float32)
    idxi = jnp.broadcast_to(idxf[:, None], (B, 128))
    idxjc = idxf.reshape(B // 128, 128)

    tc_run = pl.pallas_call(
        _tc_body,
        grid=(NBI, NBJ),
        in_specs=[
            pl.BlockSpec((BI, 128), lambda i, j: (i, 0)),
            pl.BlockSpec((BJ // 128, 128), lambda i, j: (j, 0)),
            pl.BlockSpec((BJ, D), lambda i, j: (j, 0)),
            pl.BlockSpec((BI, D), lambda i, j: (i, 0)),
            pl.BlockSpec((BI, D), lambda i, j: (i, 0)),
        ],
        out_specs=pl.BlockSpec((BI, D), lambda i, j: (i, 0)),
        out_shape=jax.ShapeDtypeStruct((B, D), jnp.float32),
        scratch_shapes=[pltpu.VMEM((BI, D), jnp.float32)],
        compiler_params=pltpu.CompilerParams(
            dimension_semantics=("parallel", "arbitrary")),
    )
    return tc_run(idxi, idxjc, val, val, sc_sel)


# speedup vs baseline: 1.1518x; 1.1137x over previous
import functools

import jax
import jax.numpy as jnp
from jax import lax
from jax.experimental import pallas as pl
from jax.experimental.pallas import tpu as pltpu
from jax.experimental.pallas import tpu_sc as plsc

NUM_NODES = 1000000
NP = NUM_NODES // 2
B = 16384
D = 64
NT = 16
G = 128
CH = B // NT
NG = CH // G
BI = 1024
BJ = 1024
NBI = B // BI
NBJ = B // BJ


def _sc_body(ns2, pidx1, val128, out, *rest):
    pidx_g = rest[0:NG]
    big1, big2, sem_g = rest[NG:]

    c = lax.axis_index("c")
    s = lax.axis_index("s")

    for j in range(NG):
        pltpu.sync_copy(pidx1.at[pl.ds(s * CH + j * G, G)], pidx_g[j])

    for j in range(NG):
        base = s * CH + j * G
        pltpu.async_copy(ns2.at[pidx_g[j]], big1, sem_g).wait()
        pltpu.sync_copy(val128.at[pl.ds(base, G)], big2)

        def addrow(row, carry):
            for k in range(G // 16):
                d = pl.ds(k * 16, 16)
                big1[row, d] = big1[row, d] + big2[row, d]
            return carry
        lax.fori_loop(0, G, addrow, 0)

        @pl.when((base // (B // 2)) == c)
        def _(base=base):
            pltpu.sync_copy(big1, out.at[pl.ds(base, G)])


def _tc_body(idxi, idxjc, valj, vali, sc, o):
    ii = idxi[...]
    vj = valj[...].astype(jnp.bfloat16)
    upd = jnp.zeros((BI, D), jnp.float32)
    for t in range(B // 128):
        m = (ii == idxjc[t:t + 1, :]).astype(jnp.bfloat16)
        upd += lax.dot_general(
            m, vj[t * 128:(t + 1) * 128, :],
            (((1,), (0,)), ((), ())),
            preferred_element_type=jnp.float32)
    vb = vali[...].astype(jnp.bfloat16).astype(jnp.float32)
    o[...] = sc[...] + upd - vb


def kernel(node_states, idx, val):
    idx32 = idx.astype(jnp.int32)
    pidx1 = idx32 >> 1
    ns2 = node_states.reshape(NP, G)
    odd = (idx32 & 1).astype(jnp.bool_)
    zero = jnp.zeros_like(val)
    val128 = jnp.where(odd[:, None],
                       jnp.concatenate([zero, val], axis=1),
                       jnp.concatenate([val, zero], axis=1))

    sc_run = pl.kernel(
        _sc_body,
        out_type=jax.ShapeDtypeStruct((B, G), jnp.float32),
        mesh=plsc.VectorSubcoreMesh(core_axis_name="c", subcore_axis_name="s"),
        scratch_types=(
            [pltpu.VMEM((G,), jnp.int32)] * NG
            + [pltpu.VMEM((G, G), jnp.float32),
               pltpu.VMEM((G, G), jnp.float32),
               pltpu.SemaphoreType.DMA]
        ),
    )
    sc128 = sc_run(ns2, pidx1, val128)
    sc_sel = jnp.where(odd[:, None], sc128[:, D:], sc128[:, :D])

    idxf = idx32.astype(jnp.float32)
    idxi = jnp.broadcast_to(idxf[:, None], (B, 128))
    idxjc = idxf.reshape(B // 128, 128)

    tc_run = pl.pallas_call(
        _tc_body,
        grid=(NBI,),
        in_specs=[
            pl.BlockSpec((BI, 128), lambda i: (i, 0)),
            pl.BlockSpec((B // 128, 128), lambda i: (0, 0)),
            pl.BlockSpec((B, D), lambda i: (0, 0)),
            pl.BlockSpec((BI, D), lambda i: (i, 0)),
            pl.BlockSpec((BI, D), lambda i: (i, 0)),
        ],
        out_specs=pl.BlockSpec((BI, D), lambda i: (i, 0)),
        out_shape=jax.ShapeDtypeStruct((B, D), jnp.float32),
        compiler_params=pltpu.CompilerParams(
            dimension_semantics=("arbitrary",)),
    )
    return tc_run(idxi, idxjc, val, val, sc_sel)
